# Initial kernel scaffold; baseline (speedup 1.0000x reference)
import jax
import jax.numpy as jnp
from jax.experimental import pallas as pl


def kernel(height_map, semantic_map, biome2mclabels):
    raise NotImplementedError("write your pallas kernel here")


# speedup vs baseline: 254.6844x; 254.6844x over previous
import jax
import jax.numpy as jnp
from jax import lax
from jax.experimental import pallas as pl
from jax.experimental.pallas import tpu as pltpu

_H = 256
_X = 512
_Y = 512
_FILL = 16
_ZB = 16


def _fill_body(hm_ref, sm_ref, tab_ref, out_ref, h_scr, sem_scr):
    zi = pl.program_id(0)

    @pl.when(zi == 0)
    def _prep():
        h_scr[...] = jnp.clip((hm_ref[0] * (_H - 1)).astype(jnp.int32), 0, _H - 1)
        sm = sm_ref[0]
        sem = jnp.zeros((_X, _Y), jnp.float32)
        for i in range(10):
            sem = jnp.where(sm == i, tab_ref[i], sem)
        sem_scr[...] = sem

    z = zi * _ZB + lax.broadcasted_iota(jnp.int32, (_ZB, _X, _Y), 0)
    diff = z - h_scr[...][None, :, :]
    mask = (diff >= 0) & (diff <= _FILL)
    out_ref[...] = jnp.where(mask, sem_scr[...][None, :, :], 0.0)


def kernel(height_map, semantic_map, biome2mclabels):
    return pl.pallas_call(
        _fill_body,
        grid=(_H // _ZB,),
        in_specs=[
            pl.BlockSpec((1, _X, _Y), lambda z: (0, 0, 0)),
            pl.BlockSpec((1, _X, _Y), lambda z: (0, 0, 0)),
            pl.BlockSpec(memory_space=pltpu.SMEM),
        ],
        out_specs=pl.BlockSpec((_ZB, _X, _Y), lambda z: (z, 0, 0)),
        out_shape=jax.ShapeDtypeStruct((_H, _X, _Y), jnp.float32),
        scratch_shapes=[
            pltpu.VMEM((_X, _Y), jnp.int32),
            pltpu.VMEM((_X, _Y), jnp.float32),
        ],
    )(height_map, semantic_map, biome2mclabels)
